# Initial kernel scaffold; baseline (speedup 1.0000x reference)
import jax
import jax.numpy as jnp
from jax.experimental import pallas as pl


def kernel(x, edge_index, W_l, b_l, W_r):
    raise NotImplementedError("write your pallas kernel here")


# speedup vs baseline: 6.2714x; 6.2714x over previous
import functools

import jax
import jax.numpy as jnp
from jax import lax
from jax.experimental import pallas as pl
from jax.experimental.pallas import tpu as pltpu
from jax.experimental.pallas import tpu_sc as plsc

N_NODES = 10000
D = 128
N_EDGES = 320000

NC = 2
NS = 16
NW = NC * NS
EDGES_PER_TILE = N_EDGES // NW
CHUNK = 80
N_CHUNKS = EDGES_PER_TILE // CHUNK
ROWS_A = 632
ROWS_LAST = N_NODES - (NS - 1) * ROWS_A
ZROWS = 8
HR = 80


def _sc_accumulate(x, src, dst):
    mesh = plsc.VectorSubcoreMesh(core_axis_name="c", subcore_axis_name="s")

    @functools.partial(
        pl.kernel,
        out_type=(
            jax.ShapeDtypeStruct((NC, N_NODES, D), jnp.float32),
            jax.ShapeDtypeStruct((NW, HR, D), jnp.float32),
        ),
        mesh=mesh,
        compiler_params=pltpu.CompilerParams(needs_layout_passes=False),
        scratch_types=[
            pltpu.VMEM_SHARED((N_NODES, D), jnp.float32),
            pltpu.VMEM((CHUNK,), jnp.int32),
            pltpu.VMEM((CHUNK,), jnp.int32),
            pltpu.VMEM((CHUNK, D), jnp.float32),
            pltpu.VMEM((HR, D), jnp.float32),
            pltpu.VMEM((ZROWS, D), jnp.float32),
            pltpu.SemaphoreType.DMA,
        ],
    )
    def sc_kernel(x_hbm, src_hbm, dst_hbm,
                  feat_out, hist_out,
                  feat_acc, src_v, dst_v, rows_v, hist_v, zb_v, sem):
        cid = lax.axis_index("c")
        sid = lax.axis_index("s")
        wid = cid * NS + sid

        zvec = jnp.zeros((16,), jnp.float32)
        for i in range(ZROWS):
            for j in range(D // 16):
                zb_v[i, pl.ds(j * 16, 16)] = zvec

        def zero_hist(i, _):
            for j in range(D // 16):
                hist_v[i, pl.ds(j * 16, 16)] = zvec
            return 0
        lax.fori_loop(0, HR, zero_hist, 0)

        row0 = sid * ROWS_A

        def zero_rows(nrows):
            def zbody(k, _):
                pltpu.sync_copy(zb_v, feat_acc.at[pl.ds(row0 + k * ZROWS, ZROWS)])
                return 0
            lax.fori_loop(0, nrows // ZROWS, zbody, 0)

        pl.when(sid < NS - 1)(lambda: zero_rows(ROWS_A))
        pl.when(sid == NS - 1)(lambda: zero_rows(ROWS_LAST))
        plsc.subcore_barrier()

        edge_base = wid * EDGES_PER_TILE
        ones16 = jnp.ones((16,), jnp.float32)

        def body(i, _):
            base = edge_base + i * CHUNK
            pltpu.sync_copy(src_hbm.at[pl.ds(base, CHUNK)], src_v)
            pltpu.sync_copy(dst_hbm.at[pl.ds(base, CHUNK)], dst_v)
            pltpu.async_copy(x_hbm.at[src_v], rows_v, sem).wait()
            pltpu.sync_copy(rows_v, feat_acc.at[dst_v], add=True)
            for k in range(CHUNK // 16):
                dv = dst_v[pl.ds(k * 16, 16)]
                hi = lax.shift_right_logical(dv, 7)
                lo = lax.bitwise_and(dv, 127)
                plsc.addupdate_scatter(hist_v, [hi, lo], ones16)
            return 0

        lax.fori_loop(0, N_CHUNKS, body, 0)
        plsc.subcore_barrier()

        pltpu.sync_copy(hist_v, hist_out.at[wid])

        def write_rows(nrows):
            pltpu.sync_copy(feat_acc.at[pl.ds(row0, nrows)],
                            feat_out.at[cid].at[pl.ds(row0, nrows)])

        pl.when(sid < NS - 1)(lambda: write_rows(ROWS_A))
        pl.when(sid == NS - 1)(lambda: write_rows(ROWS_LAST))

    return sc_kernel(x, src, dst)


def _cnt_body(h_ref, o_ref):
    o_ref[...] = jnp.sum(h_ref[...], axis=0)


def _cnt_sum(hist_p):
    return pl.pallas_call(
        _cnt_body,
        out_shape=jax.ShapeDtypeStruct((HR, D), jnp.float32),
    )(hist_p)


BR = 1000


def _tc_body(fp_ref, cnt_ref, x_ref, wl_ref, bl_ref, wr_ref, o_ref):
    feat = fp_ref[0] + fp_ref[1]
    mean = feat / jnp.maximum(cnt_ref[...], 1.0)
    dn = (((1,), (1,)), ((), ()))
    o_ref[...] = (
        lax.dot_general(mean, wl_ref[...], dn, preferred_element_type=jnp.float32)
        + lax.dot_general(x_ref[...], wr_ref[...], dn, preferred_element_type=jnp.float32)
        + bl_ref[...]
    )


def _tc_finish(feat_p, cnt, x, W_l, b_l2, W_r):
    grid = (N_NODES // BR,)
    return pl.pallas_call(
        _tc_body,
        grid=grid,
        in_specs=[
            pl.BlockSpec((NC, BR, D), lambda i: (0, i, 0)),
            pl.BlockSpec((BR, 1), lambda i: (i, 0)),
            pl.BlockSpec((BR, D), lambda i: (i, 0)),
            pl.BlockSpec((D, D), lambda i: (0, 0)),
            pl.BlockSpec((1, D), lambda i: (0, 0)),
            pl.BlockSpec((D, D), lambda i: (0, 0)),
        ],
        out_specs=pl.BlockSpec((BR, D), lambda i: (i, 0)),
        out_shape=jax.ShapeDtypeStruct((N_NODES, D), jnp.float32),
    )(feat_p, cnt, x, W_l, b_l2, W_r)


@jax.jit
def kernel(x, edge_index, W_l, b_l, W_r):
    src = edge_index[0].astype(jnp.int32)
    dst = edge_index[1].astype(jnp.int32)
    feat_p, hist_p = _sc_accumulate(x, src, dst)
    cnt = _cnt_sum(hist_p).reshape(HR * D)[:N_NODES].reshape(N_NODES, 1)
    return _tc_finish(feat_p, cnt, x, W_l, b_l.reshape(1, D), W_r)


# speedup vs baseline: 12.4560x; 1.9862x over previous
import functools

import jax
import jax.numpy as jnp
from jax import lax
from jax.experimental import pallas as pl
from jax.experimental.pallas import tpu as pltpu
from jax.experimental.pallas import tpu_sc as plsc

N_NODES = 10000
D = 128
N_EDGES = 320000

NC = 2
NS = 16
NW = NC * NS
EDGES_PER_TILE = N_EDGES // NW
CHUNK = 80
N_CHUNKS = EDGES_PER_TILE // CHUNK
ROWS_A = 632
ROWS_LAST = N_NODES - (NS - 1) * ROWS_A
ZROWS = 8
HR = 80


def _sc_accumulate(x, src, dst):
    mesh = plsc.VectorSubcoreMesh(core_axis_name="c", subcore_axis_name="s")

    @functools.partial(
        pl.kernel,
        out_type=(
            jax.ShapeDtypeStruct((NC, N_NODES, D), jnp.float32),
            jax.ShapeDtypeStruct((NW, HR, D), jnp.float32),
        ),
        mesh=mesh,
        compiler_params=pltpu.CompilerParams(needs_layout_passes=False),
        scratch_types=[
            pltpu.VMEM_SHARED((N_NODES, D), jnp.float32),
            pltpu.VMEM((EDGES_PER_TILE,), jnp.int32),
            pltpu.VMEM((EDGES_PER_TILE,), jnp.int32),
            pltpu.VMEM((CHUNK, D), jnp.float32),
            pltpu.VMEM((CHUNK, D), jnp.float32),
            pltpu.VMEM((HR, D), jnp.float32),
            pltpu.SemaphoreType.DMA,
            pltpu.SemaphoreType.DMA,
        ],
    )
    def sc_kernel(x_hbm, src_hbm, dst_hbm,
                  feat_out, hist_out,
                  feat_acc, src_all, dst_all,
                  rows_v0, rows_v1, hist_v, sem0, sem1):
        cid = lax.axis_index("c")
        sid = lax.axis_index("s")
        wid = cid * NS + sid

        zvec = jnp.zeros((16,), jnp.float32)
        for i in range(ZROWS):
            for j in range(D // 16):
                rows_v0[i, pl.ds(j * 16, 16)] = zvec

        def zero_hist(i, _):
            for j in range(D // 16):
                hist_v[i, pl.ds(j * 16, 16)] = zvec
            return 0
        lax.fori_loop(0, HR, zero_hist, 0)

        row0 = sid * ROWS_A

        def zero_rows(nrows):
            def zbody(k, _):
                pltpu.sync_copy(rows_v0.at[pl.ds(0, ZROWS)],
                                feat_acc.at[pl.ds(row0 + k * ZROWS, ZROWS)])
                return 0
            lax.fori_loop(0, nrows // ZROWS, zbody, 0)

        pl.when(sid < NS - 1)(lambda: zero_rows(ROWS_A))
        pl.when(sid == NS - 1)(lambda: zero_rows(ROWS_LAST))
        plsc.subcore_barrier()

        edge_base = wid * EDGES_PER_TILE
        ones16 = jnp.ones((16,), jnp.float32)

        pltpu.sync_copy(src_hbm.at[pl.ds(edge_base, EDGES_PER_TILE)], src_all)
        pltpu.sync_copy(dst_hbm.at[pl.ds(edge_base, EDGES_PER_TILE)], dst_all)

        def start_gather(c, rows_v, sem):
            pltpu.async_copy(x_hbm.at[src_all.at[pl.ds(c * CHUNK, CHUNK)]],
                             rows_v, sem)

        def wait_gather(rows_v, sem):
            pltpu.make_async_copy(x_hbm.at[pl.ds(0, CHUNK)], rows_v, sem).wait()

        def finish_chunk(c, rows_v, sem):
            wait_gather(rows_v, sem)
            for k in range(CHUNK // 16):
                dv = dst_all[pl.ds(c * CHUNK + k * 16, 16)]
                pltpu.sync_copy(rows_v.at[pl.ds(k * 16, 16)],
                                feat_acc.at[dv], add=True)
                hi = lax.shift_right_logical(dv, 7)
                lo = lax.bitwise_and(dv, 127)
                plsc.addupdate_scatter(hist_v, [hi, lo], ones16)

        start_gather(0, rows_v0, sem0)

        def body(j, _):
            c0 = 2 * j
            start_gather(c0 + 1, rows_v1, sem1)
            finish_chunk(c0, rows_v0, sem0)
            start_gather(c0 + 2, rows_v0, sem0)
            finish_chunk(c0 + 1, rows_v1, sem1)
            return 0

        lax.fori_loop(0, (N_CHUNKS - 1) // 2, body, 0)
        finish_chunk(N_CHUNKS - 1, rows_v0, sem0)
        plsc.subcore_barrier()

        pltpu.sync_copy(hist_v, hist_out.at[wid])

        def write_rows(nrows):
            pltpu.sync_copy(feat_acc.at[pl.ds(row0, nrows)],
                            feat_out.at[cid].at[pl.ds(row0, nrows)])

        pl.when(sid < NS - 1)(lambda: write_rows(ROWS_A))
        pl.when(sid == NS - 1)(lambda: write_rows(ROWS_LAST))

    return sc_kernel(x, src, dst)


def _cnt_body(h_ref, o_ref):
    o_ref[...] = jnp.sum(h_ref[...], axis=0)


def _cnt_sum(hist_p):
    return pl.pallas_call(
        _cnt_body,
        out_shape=jax.ShapeDtypeStruct((HR, D), jnp.float32),
    )(hist_p)


BR = 1000


def _tc_body(fp_ref, cnt_ref, x_ref, wl_ref, bl_ref, wr_ref, o_ref):
    feat = fp_ref[0] + fp_ref[1]
    mean = feat / jnp.maximum(cnt_ref[...], 1.0)
    dn = (((1,), (1,)), ((), ()))
    o_ref[...] = (
        lax.dot_general(mean, wl_ref[...], dn, preferred_element_type=jnp.float32)
        + lax.dot_general(x_ref[...], wr_ref[...], dn, preferred_element_type=jnp.float32)
        + bl_ref[...]
    )


def _tc_finish(feat_p, cnt, x, W_l, b_l2, W_r):
    grid = (N_NODES // BR,)
    return pl.pallas_call(
        _tc_body,
        grid=grid,
        in_specs=[
            pl.BlockSpec((NC, BR, D), lambda i: (0, i, 0)),
            pl.BlockSpec((BR, 1), lambda i: (i, 0)),
            pl.BlockSpec((BR, D), lambda i: (i, 0)),
            pl.BlockSpec((D, D), lambda i: (0, 0)),
            pl.BlockSpec((1, D), lambda i: (0, 0)),
            pl.BlockSpec((D, D), lambda i: (0, 0)),
        ],
        out_specs=pl.BlockSpec((BR, D), lambda i: (i, 0)),
        out_shape=jax.ShapeDtypeStruct((N_NODES, D), jnp.float32),
    )(feat_p, cnt, x, W_l, b_l2, W_r)


@jax.jit
def kernel(x, edge_index, W_l, b_l, W_r):
    src = edge_index[0].astype(jnp.int32)
    dst = edge_index[1].astype(jnp.int32)
    feat_p, hist_p = _sc_accumulate(x, src, dst)
    cnt = _cnt_sum(hist_p).reshape(HR * D)[:N_NODES].reshape(N_NODES, 1)
    return _tc_finish(feat_p, cnt, x, W_l, b_l.reshape(1, D), W_r)


# speedup vs baseline: 13.4123x; 1.0768x over previous
import functools

import jax
import jax.numpy as jnp
from jax import lax
from jax.experimental import pallas as pl
from jax.experimental.pallas import tpu as pltpu
from jax.experimental.pallas import tpu_sc as plsc

N_NODES = 10000
D = 128
N_EDGES = 320000

NC = 2
NS = 16
NW = NC * NS
EDGES_PER_TILE = N_EDGES // NW
CHUNK = 80
N_CHUNKS = EDGES_PER_TILE // CHUNK
ROWS_A = 632
ROWS_LAST = N_NODES - (NS - 1) * ROWS_A
ZROWS = 8
HR = 80


def _sc_accumulate(x, src, dst):
    mesh = plsc.VectorSubcoreMesh(core_axis_name="c", subcore_axis_name="s")

    @functools.partial(
        pl.kernel,
        out_type=(
            jax.ShapeDtypeStruct((NC, N_NODES, D), jnp.float32),
            jax.ShapeDtypeStruct((NW, HR, D), jnp.float32),
        ),
        mesh=mesh,
        compiler_params=pltpu.CompilerParams(needs_layout_passes=False),
        scratch_types=[
            pltpu.VMEM_SHARED((N_NODES, D), jnp.float32),
            pltpu.VMEM((EDGES_PER_TILE,), jnp.int32),
            pltpu.VMEM((EDGES_PER_TILE,), jnp.int32),
            pltpu.VMEM((CHUNK, D), jnp.float32),
            pltpu.VMEM((CHUNK, D), jnp.float32),
            pltpu.VMEM((HR, D), jnp.float32),
            pltpu.SemaphoreType.DMA,
            pltpu.SemaphoreType.DMA,
            pltpu.SemaphoreType.DMA,
            pltpu.SemaphoreType.DMA,
        ],
    )
    def sc_kernel(x_hbm, src_hbm, dst_hbm,
                  feat_out, hist_out,
                  feat_acc, src_all, dst_all,
                  rows_v0, rows_v1, hist_v, sem0, sem1, ssem0, ssem1):
        cid = lax.axis_index("c")
        sid = lax.axis_index("s")
        wid = cid * NS + sid

        zvec = jnp.zeros((16,), jnp.float32)
        for i in range(ZROWS):
            for j in range(D // 16):
                rows_v0[i, pl.ds(j * 16, 16)] = zvec

        def zero_hist(i, _):
            for j in range(D // 16):
                hist_v[i, pl.ds(j * 16, 16)] = zvec
            return 0
        lax.fori_loop(0, HR, zero_hist, 0)

        row0 = sid * ROWS_A

        def zero_rows(nrows):
            def zbody(k, _):
                pltpu.sync_copy(rows_v0.at[pl.ds(0, ZROWS)],
                                feat_acc.at[pl.ds(row0 + k * ZROWS, ZROWS)])
                return 0
            lax.fori_loop(0, nrows // ZROWS, zbody, 0)

        pl.when(sid < NS - 1)(lambda: zero_rows(ROWS_A))
        pl.when(sid == NS - 1)(lambda: zero_rows(ROWS_LAST))
        plsc.subcore_barrier()

        edge_base = wid * EDGES_PER_TILE
        ones16 = jnp.ones((16,), jnp.float32)

        pltpu.sync_copy(src_hbm.at[pl.ds(edge_base, EDGES_PER_TILE)], src_all)
        pltpu.sync_copy(dst_hbm.at[pl.ds(edge_base, EDGES_PER_TILE)], dst_all)

        def start_gather(c, rows_v, sem):
            pltpu.async_copy(x_hbm.at[src_all.at[pl.ds(c * CHUNK, CHUNK)]],
                             rows_v, sem)

        def wait_gather(rows_v, sem):
            pltpu.make_async_copy(x_hbm.at[pl.ds(0, CHUNK)], rows_v, sem).wait()

        def finish_chunk(c, rows_v, sem, ssem):
            wait_gather(rows_v, sem)
            pend = []
            for k in range(CHUNK // 16):
                dv = dst_all[pl.ds(c * CHUNK + k * 16, 16)]
                pend.append(pltpu.async_copy(rows_v.at[pl.ds(k * 16, 16)],
                                             feat_acc.at[dv], ssem, add=True))
                hi = lax.shift_right_logical(dv, 7)
                lo = lax.bitwise_and(dv, 127)
                plsc.addupdate_scatter(hist_v, [hi, lo], ones16)
            for d in pend:
                d.wait()

        start_gather(0, rows_v0, sem0)

        def body(j, _):
            c0 = 2 * j
            start_gather(c0 + 1, rows_v1, sem1)
            finish_chunk(c0, rows_v0, sem0, ssem0)
            start_gather(c0 + 2, rows_v0, sem0)
            finish_chunk(c0 + 1, rows_v1, sem1, ssem1)
            return 0

        lax.fori_loop(0, (N_CHUNKS - 1) // 2, body, 0)
        finish_chunk(N_CHUNKS - 1, rows_v0, sem0, ssem0)
        plsc.subcore_barrier()

        pltpu.sync_copy(hist_v, hist_out.at[wid])

        def write_rows(nrows):
            pltpu.sync_copy(feat_acc.at[pl.ds(row0, nrows)],
                            feat_out.at[cid].at[pl.ds(row0, nrows)])

        pl.when(sid < NS - 1)(lambda: write_rows(ROWS_A))
        pl.when(sid == NS - 1)(lambda: write_rows(ROWS_LAST))

    return sc_kernel(x, src, dst)


def _cnt_body(h_ref, o_ref):
    o_ref[...] = jnp.sum(h_ref[...], axis=0)


def _cnt_sum(hist_p):
    return pl.pallas_call(
        _cnt_body,
        out_shape=jax.ShapeDtypeStruct((HR, D), jnp.float32),
    )(hist_p)


BR = 1000


def _tc_body(fp_ref, cnt_ref, x_ref, wl_ref, bl_ref, wr_ref, o_ref):
    feat = fp_ref[0] + fp_ref[1]
    mean = feat / jnp.maximum(cnt_ref[...], 1.0)
    dn = (((1,), (1,)), ((), ()))
    o_ref[...] = (
        lax.dot_general(mean, wl_ref[...], dn, preferred_element_type=jnp.float32)
        + lax.dot_general(x_ref[...], wr_ref[...], dn, preferred_element_type=jnp.float32)
        + bl_ref[...]
    )


def _tc_finish(feat_p, cnt, x, W_l, b_l2, W_r):
    grid = (N_NODES // BR,)
    return pl.pallas_call(
        _tc_body,
        grid=grid,
        in_specs=[
            pl.BlockSpec((NC, BR, D), lambda i: (0, i, 0)),
            pl.BlockSpec((BR, 1), lambda i: (i, 0)),
            pl.BlockSpec((BR, D), lambda i: (i, 0)),
            pl.BlockSpec((D, D), lambda i: (0, 0)),
            pl.BlockSpec((1, D), lambda i: (0, 0)),
            pl.BlockSpec((D, D), lambda i: (0, 0)),
        ],
        out_specs=pl.BlockSpec((BR, D), lambda i: (i, 0)),
        out_shape=jax.ShapeDtypeStruct((N_NODES, D), jnp.float32),
    )(feat_p, cnt, x, W_l, b_l2, W_r)


@jax.jit
def kernel(x, edge_index, W_l, b_l, W_r):
    src = edge_index[0].astype(jnp.int32)
    dst = edge_index[1].astype(jnp.int32)
    feat_p, hist_p = _sc_accumulate(x, src, dst)
    cnt = _cnt_sum(hist_p).reshape(HR * D)[:N_NODES].reshape(N_NODES, 1)
    return _tc_finish(feat_p, cnt, x, W_l, b_l.reshape(1, D), W_r)


# speedup vs baseline: 14.4747x; 1.0792x over previous
import functools

import jax
import jax.numpy as jnp
from jax import lax
from jax.experimental import pallas as pl
from jax.experimental.pallas import tpu as pltpu
from jax.experimental.pallas import tpu_sc as plsc

N_NODES = 10000
D = 128
N_EDGES = 320000

NC = 2
NS = 16
NW = NC * NS
EDGES_PER_TILE = N_EDGES // NW
CHUNK = 80
N_CHUNKS = EDGES_PER_TILE // CHUNK
SEGMENTS = ((0, 60), (60, 60), (120, 5))
SEG_MAX = 60
ROWS_A = 632
ROWS_LAST = N_NODES - (NS - 1) * ROWS_A
ZROWS = 8
HR = 80


def _sc_accumulate(x, src, dst):
    mesh = plsc.VectorSubcoreMesh(core_axis_name="c", subcore_axis_name="s")

    @functools.partial(
        pl.kernel,
        out_type=(
            jax.ShapeDtypeStruct((NC, N_NODES, D), jnp.float32),
            jax.ShapeDtypeStruct((NW, HR, D), jnp.float32),
        ),
        mesh=mesh,
        compiler_params=pltpu.CompilerParams(needs_layout_passes=False),
        scratch_types=[
            pltpu.VMEM_SHARED((N_NODES, D), jnp.float32),
            pltpu.VMEM((SEG_MAX * CHUNK,), jnp.int32),
            pltpu.VMEM((SEG_MAX * CHUNK,), jnp.int32),
            pltpu.VMEM((CHUNK, D), jnp.float32),
            pltpu.VMEM((CHUNK, D), jnp.float32),
            pltpu.VMEM((CHUNK, D), jnp.float32),
            pltpu.VMEM((HR, D), jnp.float32),
            pltpu.SemaphoreType.DMA,
            pltpu.SemaphoreType.DMA,
            pltpu.SemaphoreType.DMA,
            pltpu.SemaphoreType.DMA,
            pltpu.SemaphoreType.DMA,
            pltpu.SemaphoreType.DMA,
        ],
    )
    def sc_kernel(x_hbm, src_hbm, dst_hbm,
                  feat_out, hist_out,
                  feat_acc, src_all, dst_all,
                  rows_v0, rows_v1, rows_v2, hist_v,
                  sem0, sem1, sem2, ssem0, ssem1, ssem2):
        cid = lax.axis_index("c")
        sid = lax.axis_index("s")
        wid = cid * NS + sid

        zvec = jnp.zeros((16,), jnp.float32)
        for i in range(ZROWS):
            for j in range(D // 16):
                rows_v0[i, pl.ds(j * 16, 16)] = zvec

        def zero_hist(i, _):
            for j in range(D // 16):
                hist_v[i, pl.ds(j * 16, 16)] = zvec
            return 0
        lax.fori_loop(0, HR, zero_hist, 0)

        row0 = sid * ROWS_A

        def zero_rows(nrows):
            def zbody(k, _):
                pltpu.sync_copy(rows_v0.at[pl.ds(0, ZROWS)],
                                feat_acc.at[pl.ds(row0 + k * ZROWS, ZROWS)])
                return 0
            lax.fori_loop(0, nrows // ZROWS, zbody, 0)

        pl.when(sid < NS - 1)(lambda: zero_rows(ROWS_A))
        pl.when(sid == NS - 1)(lambda: zero_rows(ROWS_LAST))
        plsc.subcore_barrier()

        edge_base = wid * EDGES_PER_TILE
        ones16 = jnp.ones((16,), jnp.float32)
        bufs = ((rows_v0, sem0, ssem0), (rows_v1, sem1, ssem1),
                (rows_v2, sem2, ssem2))

        def start_gather(off, b):
            rows_v, sem, _ = bufs[b]
            pltpu.async_copy(x_hbm.at[src_all.at[pl.ds(off * CHUNK, CHUNK)]],
                             rows_v, sem)

        def finish_chunk(off, b, nxt_off, start_next):
            rows_v, sem, ssem = bufs[b]
            pltpu.make_async_copy(x_hbm.at[pl.ds(0, CHUNK)], rows_v, sem).wait()
            pend = []
            dvs = []
            for k in range(CHUNK // 16):
                dv = dst_all[pl.ds(off * CHUNK + k * 16, 16)]
                dvs.append(dv)
                pend.append(pltpu.async_copy(rows_v.at[pl.ds(k * 16, 16)],
                                             feat_acc.at[dv], ssem, add=True))
            if start_next:
                start_gather(nxt_off, (b + 2) % 3)
            for dv in dvs:
                hi = lax.shift_right_logical(dv, 7)
                lo = lax.bitwise_and(dv, 127)
                plsc.addupdate_scatter(hist_v, [hi, lo], ones16)
            for d in pend:
                d.wait()

        for base, n in SEGMENTS:
            pltpu.sync_copy(
                src_hbm.at[pl.ds(edge_base + base * CHUNK, n * CHUNK)],
                src_all.at[pl.ds(0, n * CHUNK)])
            pltpu.sync_copy(
                dst_hbm.at[pl.ds(edge_base + base * CHUNK, n * CHUNK)],
                dst_all.at[pl.ds(0, n * CHUNK)])
            start_gather(0, 0)
            start_gather(1, 1)
            k3 = (n - 2) // 3
            rem = (n - 2) - 3 * k3

            def body(j, _, k3=k3):
                for t in range(3):
                    off = 3 * j + t
                    finish_chunk(off, t, off + 2, True)
                return 0

            lax.fori_loop(0, k3, body, 0)
            for t in range(2 + rem):
                off = 3 * k3 + t
                finish_chunk(off, t % 3, off + 2, off + 2 <= n - 1)
        plsc.subcore_barrier()

        pltpu.sync_copy(hist_v, hist_out.at[wid])

        def write_rows(nrows):
            pltpu.sync_copy(feat_acc.at[pl.ds(row0, nrows)],
                            feat_out.at[cid].at[pl.ds(row0, nrows)])

        pl.when(sid < NS - 1)(lambda: write_rows(ROWS_A))
        pl.when(sid == NS - 1)(lambda: write_rows(ROWS_LAST))

    return sc_kernel(x, src, dst)


def _cnt_body(h_ref, o_ref):
    o_ref[...] = jnp.sum(h_ref[...], axis=0)


def _cnt_sum(hist_p):
    return pl.pallas_call(
        _cnt_body,
        out_shape=jax.ShapeDtypeStruct((HR, D), jnp.float32),
    )(hist_p)


BR = 1000


def _tc_body(fp_ref, cnt_ref, x_ref, wl_ref, bl_ref, wr_ref, o_ref):
    feat = fp_ref[0] + fp_ref[1]
    mean = feat / jnp.maximum(cnt_ref[...], 1.0)
    dn = (((1,), (1,)), ((), ()))
    o_ref[...] = (
        lax.dot_general(mean, wl_ref[...], dn, preferred_element_type=jnp.float32)
        + lax.dot_general(x_ref[...], wr_ref[...], dn, preferred_element_type=jnp.float32)
        + bl_ref[...]
    )


def _tc_finish(feat_p, cnt, x, W_l, b_l2, W_r):
    grid = (N_NODES // BR,)
    return pl.pallas_call(
        _tc_body,
        grid=grid,
        in_specs=[
            pl.BlockSpec((NC, BR, D), lambda i: (0, i, 0)),
            pl.BlockSpec((BR, 1), lambda i: (i, 0)),
            pl.BlockSpec((BR, D), lambda i: (i, 0)),
            pl.BlockSpec((D, D), lambda i: (0, 0)),
            pl.BlockSpec((1, D), lambda i: (0, 0)),
            pl.BlockSpec((D, D), lambda i: (0, 0)),
        ],
        out_specs=pl.BlockSpec((BR, D), lambda i: (i, 0)),
        out_shape=jax.ShapeDtypeStruct((N_NODES, D), jnp.float32),
    )(feat_p, cnt, x, W_l, b_l2, W_r)


@jax.jit
def kernel(x, edge_index, W_l, b_l, W_r):
    src = edge_index[0].astype(jnp.int32)
    dst = edge_index[1].astype(jnp.int32)
    feat_p, hist_p = _sc_accumulate(x, src, dst)
    cnt = _cnt_sum(hist_p).reshape(HR * D)[:N_NODES].reshape(N_NODES, 1)
    return _tc_finish(feat_p, cnt, x, W_l, b_l.reshape(1, D), W_r)


# speedup vs baseline: 15.4390x; 1.0666x over previous
import functools

import jax
import jax.numpy as jnp
from jax import lax
from jax.experimental import pallas as pl
from jax.experimental.pallas import tpu as pltpu
from jax.experimental.pallas import tpu_sc as plsc

N_NODES = 10000
D = 128
N_EDGES = 320000

NC = 2
NS = 16
NW = NC * NS
EDGES_PER_TILE = N_EDGES // NW
CHUNK = 80
N_CHUNKS = EDGES_PER_TILE // CHUNK
SEGMENTS = ((0, 60), (60, 60), (120, 5))
SEG_MAX = 60
ROWS_A = 632
ROWS_LAST = N_NODES - (NS - 1) * ROWS_A
ZROWS = 8
HR = 80


def _sc_accumulate(x, ei):
    mesh = plsc.VectorSubcoreMesh(core_axis_name="c", subcore_axis_name="s")

    @functools.partial(
        pl.kernel,
        out_type=(
            jax.ShapeDtypeStruct((NC, N_NODES, D), jnp.float32),
            jax.ShapeDtypeStruct((NW, HR, D), jnp.float32),
        ),
        mesh=mesh,
        compiler_params=pltpu.CompilerParams(needs_layout_passes=False),
        scratch_types=[
            pltpu.VMEM_SHARED((N_NODES, D), jnp.float32),
            pltpu.VMEM((SEG_MAX * CHUNK,), jnp.int32),
            pltpu.VMEM((SEG_MAX * CHUNK,), jnp.int32),
            pltpu.VMEM((CHUNK, D), jnp.float32),
            pltpu.VMEM((CHUNK, D), jnp.float32),
            pltpu.VMEM((CHUNK, D), jnp.float32),
            pltpu.VMEM((HR, D), jnp.float32),
            pltpu.SemaphoreType.DMA,
            pltpu.SemaphoreType.DMA,
            pltpu.SemaphoreType.DMA,
            pltpu.SemaphoreType.DMA,
            pltpu.SemaphoreType.DMA,
            pltpu.SemaphoreType.DMA,
        ],
    )
    def sc_kernel(x_hbm, ei_hbm,
                  feat_out, hist_out,
                  feat_acc, src_all, dst_all,
                  rows_v0, rows_v1, rows_v2, hist_v,
                  sem0, sem1, sem2, ssem0, ssem1, ssem2):
        cid = lax.axis_index("c")
        sid = lax.axis_index("s")
        wid = cid * NS + sid

        zvec = jnp.zeros((16,), jnp.float32)
        for i in range(ZROWS):
            for j in range(D // 16):
                rows_v0[i, pl.ds(j * 16, 16)] = zvec

        def zero_hist(i, _):
            for j in range(D // 16):
                hist_v[i, pl.ds(j * 16, 16)] = zvec
            return 0
        lax.fori_loop(0, HR, zero_hist, 0)

        row0 = sid * ROWS_A

        def zero_rows(nrows):
            def zbody(k, _):
                pltpu.sync_copy(rows_v0.at[pl.ds(0, ZROWS)],
                                feat_acc.at[pl.ds(row0 + k * ZROWS, ZROWS)])
                return 0
            lax.fori_loop(0, nrows // ZROWS, zbody, 0)

        pl.when(sid < NS - 1)(lambda: zero_rows(ROWS_A))
        pl.when(sid == NS - 1)(lambda: zero_rows(ROWS_LAST))
        plsc.subcore_barrier()

        edge_base = wid * EDGES_PER_TILE
        ones16 = jnp.ones((16,), jnp.float32)
        bufs = ((rows_v0, sem0, ssem0), (rows_v1, sem1, ssem1),
                (rows_v2, sem2, ssem2))

        def start_gather(off, b):
            rows_v, sem, _ = bufs[b]
            pltpu.async_copy(x_hbm.at[src_all.at[pl.ds(off * CHUNK, CHUNK)]],
                             rows_v, sem)

        def finish_chunk(off, b, nxt_off, start_next):
            rows_v, sem, ssem = bufs[b]
            pltpu.make_async_copy(x_hbm.at[pl.ds(0, CHUNK)], rows_v, sem).wait()
            pend = []
            dvs = []
            for k in range(CHUNK // 16):
                dv = dst_all[pl.ds(off * CHUNK + k * 16, 16)]
                dvs.append(dv)
                pend.append(pltpu.async_copy(rows_v.at[pl.ds(k * 16, 16)],
                                             feat_acc.at[dv], ssem, add=True))
            if start_next:
                start_gather(nxt_off, (b + 2) % 3)
            for dv in dvs:
                hi = lax.shift_right_logical(dv, 7)
                lo = lax.bitwise_and(dv, 127)
                plsc.addupdate_scatter(hist_v, [hi, lo], ones16)
            for d in pend:
                d.wait()

        for base, n in SEGMENTS:
            pltpu.sync_copy(
                ei_hbm.at[pl.ds(edge_base + base * CHUNK, n * CHUNK)],
                src_all.at[pl.ds(0, n * CHUNK)])
            pltpu.sync_copy(
                ei_hbm.at[pl.ds(N_EDGES + edge_base + base * CHUNK, n * CHUNK)],
                dst_all.at[pl.ds(0, n * CHUNK)])
            start_gather(0, 0)
            start_gather(1, 1)
            k3 = (n - 2) // 3
            rem = (n - 2) - 3 * k3

            def body(j, _, k3=k3):
                for t in range(3):
                    off = 3 * j + t
                    finish_chunk(off, t, off + 2, True)
                return 0

            lax.fori_loop(0, k3, body, 0)
            for t in range(2 + rem):
                off = 3 * k3 + t
                finish_chunk(off, t % 3, off + 2, off + 2 <= n - 1)
        plsc.subcore_barrier()

        pltpu.sync_copy(hist_v, hist_out.at[wid])

        def write_rows(nrows):
            pltpu.sync_copy(feat_acc.at[pl.ds(row0, nrows)],
                            feat_out.at[cid].at[pl.ds(row0, nrows)])

        pl.when(sid < NS - 1)(lambda: write_rows(ROWS_A))
        pl.when(sid == NS - 1)(lambda: write_rows(ROWS_LAST))

    return sc_kernel(x, ei)


def _cnt_body(h_ref, o_ref):
    o_ref[...] = jnp.sum(h_ref[...], axis=0)


def _cnt_sum(hist_p):
    return pl.pallas_call(
        _cnt_body,
        out_shape=jax.ShapeDtypeStruct((HR, D), jnp.float32),
    )(hist_p)


BR = 1000


def _tc_body(fp_ref, cnt_ref, x_ref, wl_ref, bl_ref, wr_ref, o_ref):
    feat = fp_ref[0] + fp_ref[1]
    mean = feat / jnp.maximum(cnt_ref[...], 1.0)
    dn = (((1,), (1,)), ((), ()))
    o_ref[...] = (
        lax.dot_general(mean, wl_ref[...], dn, preferred_element_type=jnp.float32)
        + lax.dot_general(x_ref[...], wr_ref[...], dn, preferred_element_type=jnp.float32)
        + bl_ref[...]
    )


def _tc_finish(feat_p, cnt, x, W_l, b_l2, W_r):
    grid = (N_NODES // BR,)
    return pl.pallas_call(
        _tc_body,
        grid=grid,
        in_specs=[
            pl.BlockSpec((NC, BR, D), lambda i: (0, i, 0)),
            pl.BlockSpec((BR, 1), lambda i: (i, 0)),
            pl.BlockSpec((BR, D), lambda i: (i, 0)),
            pl.BlockSpec((D, D), lambda i: (0, 0)),
            pl.BlockSpec((1, D), lambda i: (0, 0)),
            pl.BlockSpec((D, D), lambda i: (0, 0)),
        ],
        out_specs=pl.BlockSpec((BR, D), lambda i: (i, 0)),
        out_shape=jax.ShapeDtypeStruct((N_NODES, D), jnp.float32),
    )(feat_p, cnt, x, W_l, b_l2, W_r)


@jax.jit
def kernel(x, edge_index, W_l, b_l, W_r):
    ei_flat = edge_index.astype(jnp.int32).reshape(2 * N_EDGES)
    feat_p, hist_p = _sc_accumulate(x, ei_flat)
    cnt = _cnt_sum(hist_p).reshape(HR * D)[:N_NODES].reshape(N_NODES, 1)
    return _tc_finish(feat_p, cnt, x, W_l, b_l.reshape(1, D), W_r)


# speedup vs baseline: 16.2800x; 1.0545x over previous
import functools

import jax
import jax.numpy as jnp
from jax import lax
from jax.experimental import pallas as pl
from jax.experimental.pallas import tpu as pltpu
from jax.experimental.pallas import tpu_sc as plsc

N_NODES = 10000
D = 128
N_EDGES = 320000

NC = 2
NS = 16
NW = NC * NS
EDGES_PER_TILE = N_EDGES // NW
CHUNK = 80
N_CHUNKS = EDGES_PER_TILE // CHUNK
SEGMENTS = ((0, 60), (60, 60), (120, 5))
SEG_MAX = 60
ROWS_A = 632
ROWS_LAST = N_NODES - (NS - 1) * ROWS_A
ZROWS = 8
HR = 80


def _sc_accumulate(x, ei):
    mesh = plsc.VectorSubcoreMesh(core_axis_name="c", subcore_axis_name="s")

    @functools.partial(
        pl.kernel,
        out_type=(
            jax.ShapeDtypeStruct((NC, N_NODES, D), jnp.float32),
            jax.ShapeDtypeStruct((NW, HR, D), jnp.float32),
        ),
        mesh=mesh,
        compiler_params=pltpu.CompilerParams(needs_layout_passes=False),
        scratch_types=[
            pltpu.VMEM_SHARED((N_NODES, D), jnp.float32),
            pltpu.VMEM((SEG_MAX * CHUNK,), jnp.int32),
            pltpu.VMEM((SEG_MAX * CHUNK,), jnp.int32),
            pltpu.VMEM((CHUNK, D), jnp.float32),
            pltpu.VMEM((CHUNK, D), jnp.float32),
            pltpu.VMEM((CHUNK, D), jnp.float32),
            pltpu.VMEM((HR, D), jnp.float32),
            pltpu.SemaphoreType.DMA,
            pltpu.SemaphoreType.DMA,
            pltpu.SemaphoreType.DMA,
            pltpu.SemaphoreType.DMA,
            pltpu.SemaphoreType.DMA,
            pltpu.SemaphoreType.DMA,
        ],
    )
    def sc_kernel(x_hbm, ei_hbm,
                  feat_out, hist_out,
                  feat_acc, src_all, dst_all,
                  rows_v0, rows_v1, rows_v2, hist_v,
                  sem0, sem1, sem2, ssem0, ssem1, ssem2):
        cid = lax.axis_index("c")
        sid = lax.axis_index("s")
        wid = cid * NS + sid

        zvec = jnp.zeros((16,), jnp.float32)
        for i in range(ZROWS):
            for j in range(D // 16):
                rows_v0[i, pl.ds(j * 16, 16)] = zvec

        def zero_hist(i, _):
            for j in range(D // 16):
                hist_v[i, pl.ds(j * 16, 16)] = zvec
            return 0
        lax.fori_loop(0, HR, zero_hist, 0)

        row0 = sid * ROWS_A

        def zero_rows(nrows):
            def zbody(k, _):
                pltpu.sync_copy(rows_v0.at[pl.ds(0, ZROWS)],
                                feat_acc.at[pl.ds(row0 + k * ZROWS, ZROWS)])
                return 0
            lax.fori_loop(0, nrows // ZROWS, zbody, 0)

        pl.when(sid < NS - 1)(lambda: zero_rows(ROWS_A))
        pl.when(sid == NS - 1)(lambda: zero_rows(ROWS_LAST))
        plsc.subcore_barrier()

        edge_base = wid * EDGES_PER_TILE
        ones16 = jnp.ones((16,), jnp.float32)
        bufs = ((rows_v0, sem0, ssem0), (rows_v1, sem1, ssem1),
                (rows_v2, sem2, ssem2))

        def start_gather(off, b):
            rows_v, sem, _ = bufs[b]
            pltpu.async_copy(x_hbm.at[src_all.at[pl.ds(off * CHUNK, CHUNK)]],
                             rows_v, sem)

        def finish_chunk(off, b, nxt_off, start_next):
            rows_v, sem, ssem = bufs[b]
            pltpu.make_async_copy(x_hbm.at[pl.ds(0, CHUNK)], rows_v, sem).wait()
            pend = []
            dvs = []
            for k in range(CHUNK // 16):
                dv = dst_all[pl.ds(off * CHUNK + k * 16, 16)]
                dvs.append(dv)
                pend.append(pltpu.async_copy(rows_v.at[pl.ds(k * 16, 16)],
                                             feat_acc.at[dv], ssem, add=True))
            if start_next:
                start_gather(nxt_off, (b + 2) % 3)
            for dv in dvs:
                hi = lax.shift_right_logical(dv, 7)
                lo = lax.bitwise_and(dv, 127)
                plsc.addupdate_scatter(hist_v, [hi, lo], ones16)
            for d in pend:
                d.wait()

        for base, n in SEGMENTS:
            pltpu.sync_copy(
                ei_hbm.at[pl.ds(edge_base + base * CHUNK, n * CHUNK)],
                src_all.at[pl.ds(0, n * CHUNK)])
            pltpu.sync_copy(
                ei_hbm.at[pl.ds(N_EDGES + edge_base + base * CHUNK, n * CHUNK)],
                dst_all.at[pl.ds(0, n * CHUNK)])
            start_gather(0, 0)
            start_gather(1, 1)
            k3 = (n - 2) // 3
            rem = (n - 2) - 3 * k3

            def body(j, _, k3=k3):
                for t in range(3):
                    off = 3 * j + t
                    finish_chunk(off, t, off + 2, True)
                return 0

            lax.fori_loop(0, k3, body, 0)
            for t in range(2 + rem):
                off = 3 * k3 + t
                finish_chunk(off, t % 3, off + 2, off + 2 <= n - 1)
        plsc.subcore_barrier()

        pltpu.sync_copy(hist_v, hist_out.at[wid])

        def write_rows(nrows):
            pltpu.sync_copy(feat_acc.at[pl.ds(row0, nrows)],
                            feat_out.at[cid].at[pl.ds(row0, nrows)])

        pl.when(sid < NS - 1)(lambda: write_rows(ROWS_A))
        pl.when(sid == NS - 1)(lambda: write_rows(ROWS_LAST))

    return sc_kernel(x, ei)


BR = 1024


def _tc_body(fp_ref, h_ref, p_ref, m_ref, x_ref, wl_ref, bl_ref, wr_ref, o_ref):
    feat = fp_ref[0] + fp_ref[1]
    dn0 = (((1,), (0,)), ((), ()))
    cnt8 = jnp.sum(h_ref[...], axis=0)
    tmp = lax.dot_general(p_ref[...], cnt8, dn0, preferred_element_type=jnp.float32)
    cnt = jnp.sum(tmp * m_ref[...], axis=1, keepdims=True)
    mean = feat / jnp.maximum(cnt, 1.0)
    dn = (((1,), (1,)), ((), ()))
    o_ref[...] = (
        lax.dot_general(mean, wl_ref[...], dn, preferred_element_type=jnp.float32)
        + lax.dot_general(x_ref[...], wr_ref[...], dn, preferred_element_type=jnp.float32)
        + bl_ref[...]
    )


def _tc_finish(feat_p, hist_p, P, M, x, W_l, b_l2, W_r):
    grid = (pl.cdiv(N_NODES, BR),)
    return pl.pallas_call(
        _tc_body,
        grid=grid,
        in_specs=[
            pl.BlockSpec((NC, BR, D), lambda i: (0, i, 0)),
            pl.BlockSpec((NW, BR // D, D), lambda i: (0, i, 0)),
            pl.BlockSpec((BR, BR // D), lambda i: (0, 0)),
            pl.BlockSpec((BR, D), lambda i: (0, 0)),
            pl.BlockSpec((BR, D), lambda i: (i, 0)),
            pl.BlockSpec((D, D), lambda i: (0, 0)),
            pl.BlockSpec((1, D), lambda i: (0, 0)),
            pl.BlockSpec((D, D), lambda i: (0, 0)),
        ],
        out_specs=pl.BlockSpec((BR, D), lambda i: (i, 0)),
        out_shape=jax.ShapeDtypeStruct((N_NODES, D), jnp.float32),
    )(feat_p, hist_p, P, M, x, W_l, b_l2, W_r)


@jax.jit
def kernel(x, edge_index, W_l, b_l, W_r):
    ei_flat = edge_index.astype(jnp.int32).reshape(2 * N_EDGES)
    feat_p, hist_p = _sc_accumulate(x, ei_flat)
    ar = jnp.arange(BR, dtype=jnp.int32)[:, None]
    P = (ar // D == jnp.arange(BR // D, dtype=jnp.int32)[None, :]).astype(jnp.float32)
    M = (ar % D == jnp.arange(D, dtype=jnp.int32)[None, :]).astype(jnp.float32)
    return _tc_finish(feat_p, hist_p, P, M, x, W_l, b_l.reshape(1, D), W_r)
